# baseline (speedup 1.0000x reference)
import dataclasses
import functools

import jax
import jax.numpy as jnp
from jax import lax
from jax.experimental import pallas as pl
from jax.experimental.pallas import tpu as pltpu
from jax.experimental.pallas import tpu_sc as plsc

T = 2048
G = 16
NLANES = 16
NWORKERS = 32
CUNROLL = 8


def _splat(vec, i):
    idx = jnp.full((NLANES, 1), i, dtype=jnp.int32)
    dn = lax.GatherDimensionNumbers(
        offset_dims=(), collapsed_slice_dims=(0,), start_index_map=(0,))
    return lax.gather(vec, idx, dn, slice_sizes=(1,),
                      mode=lax.GatherScatterMode.PROMISE_IN_BOUNDS)


def _make_sc_resample(B, Lmax, C):
    N = B * T
    mesh = plsc.VectorSubcoreMesh(core_axis_name="c", subcore_axis_name="s")
    rows_per_worker = N // NWORKERS
    chpw = rows_per_worker // G
    wpb = NWORKERS // B

    cp = pltpu.CompilerParams()
    if "needs_layout_passes" in pltpu.CompilerParams.__dataclass_fields__:
        cp = dataclasses.replace(cp, needs_layout_passes=False)

    @functools.partial(
        pl.kernel,
        mesh=mesh,
        compiler_params=cp,
        out_type=jax.ShapeDtypeStruct((N, C), jnp.float32),
        scratch_types=[
            pltpu.VMEM((1, NLANES), jnp.int32),
            pltpu.VMEM((2 * G, C), jnp.float32),
            pltpu.VMEM((2 * G, C), jnp.float32),
            pltpu.VMEM((G, C), jnp.float32),
            pltpu.VMEM((G, C), jnp.float32),
            pltpu.SemaphoreType.DMA,
            pltpu.SemaphoreType.DMA,
            pltpu.SemaphoreType.DMA,
            pltpu.SemaphoreType.DMA,
        ],
    )
    def sc_resample(x_hbm, l_hbm, out_hbm,
                    lv, win_a, win_b, o_a, o_b,
                    sg0_a, sg0_b, so_a, so_b):
        win = (win_a, win_b)
        ov = (o_a, o_b)
        sg0 = (sg0_a, sg0_b)
        so = (so_a, so_b)

        wid = lax.axis_index("s") * 2 + lax.axis_index("c")
        row0 = wid * rows_per_worker

        pltpu.sync_copy(l_hbm, lv)
        lvv = lv[0, :]

        iota = lax.iota(jnp.int32, NLANES)
        iota_f = iota.astype(jnp.float32)

        def chunk_math(bb, k):
            L = _splat(lvv, bb)
            Lf = L.astype(jnp.float32)
            scale = Lf * (1.0 / float(T))
            j0 = (wid % wpb) * rows_per_worker + k * G
            jv = j0.astype(jnp.float32) + iota_f
            src = (jv + 0.5) * scale - 0.5
            src = jnp.minimum(jnp.maximum(src, 0.0), Lf - 1.0)
            i0 = src.astype(jnp.int32)
            i1 = jnp.minimum(i0 + 1, L - 1)
            w = src - i0.astype(jnp.float32)
            return i0, i1, w

        def fire(bb, roff, k, s):
            i0, i1, _ = chunk_math(bb, k)
            pltpu.make_async_copy(x_hbm.at[roff + i0],
                                  win[s].at[pl.ds(0, G), :], sg0[s]).start()
            pltpu.make_async_copy(x_hbm.at[roff + i1],
                                  win[s].at[pl.ds(G, G), :], sg0[s]).start()

        def wait_fill(s):
            pltpu.make_async_copy(x_hbm.at[iota],
                                  win[s].at[pl.ds(0, G), :], sg0[s]).wait()
            pltpu.make_async_copy(x_hbm.at[iota],
                                  win[s].at[pl.ds(G, G), :], sg0[s]).wait()

        def out_copy(k, s):
            return pltpu.make_async_copy(
                ov[s], out_hbm.at[pl.ds(row0 + k * G, G), :], so[s])

        bb = wid // wpb
        roff = bb * Lmax

        fire(bb, roff, 0, 0)

        @pl.loop(0, chpw, step=2)
        def _(k0):
            for slot in range(2):
                k = k0 + slot
                s, ns = slot, 1 - slot

                @pl.when(k + 1 < chpw)
                def _():
                    fire(bb, roff, k + 1, ns)

                wait_fill(s)

                @pl.when(k >= 2)
                def _():
                    out_copy(k, s).wait()

                _, _, w = chunk_math(bb, k)
                for r in range(G):
                    wspl = _splat(w, r)

                    @pl.loop(0, C, step=NLANES * CUNROLL)
                    def _(cc):
                        for u in range(CUNROLL):
                            sl = pl.ds(cc + u * NLANES, NLANES)
                            a = win[s][r, sl]
                            b2 = win[s][G + r, sl]
                            ov[s][r, sl] = a + wspl * (b2 - a)

                out_copy(k, s).start()

        out_copy(chpw - 2, 0).wait()
        out_copy(chpw - 1, 1).wait()

    return sc_resample


def kernel(x, lengths):
    B, Lmax, C = x.shape
    x2 = x.reshape(B * Lmax, C)
    lp = jnp.pad(lengths, (0, NLANES - B)).reshape(1, NLANES)
    out2 = _make_sc_resample(B, Lmax, C)(x2, lp)
    return out2.reshape(B, T, C)
